# baseline (speedup 1.0000x reference)
import functools

import jax
import jax.numpy as jnp
from jax import lax
from jax.experimental import pallas as pl
from jax.experimental.pallas import tpu as pltpu
from jax.experimental.pallas import tpu_sc as plsc


def _sc_scalar_lookup(blob, B, V, D):
    mesh = plsc.ScalarSubcoreMesh(axis_name="c", num_cores=1)

    @functools.partial(
        pl.kernel,
        out_type=jax.ShapeDtypeStruct((B * D,), jnp.int32),
        mesh=mesh,
        compiler_params=pltpu.CompilerParams(needs_layout_passes=False),
        scratch_types=[
            pltpu.SMEM((B + V * D,), jnp.int32),
            pltpu.SMEM((B * D,), jnp.int32),
            pltpu.SemaphoreType.DMA,
        ],
    )
    def body(blob_hbm, out_hbm, blob_s, out_s, sem):
        pltpu.sync_copy(blob_hbm, blob_s)
        for b in range(B):
            row = blob_s[b]
            for d in range(D):
                out_s[b * D + d] = blob_s[B + row * D + d]
        pltpu.sync_copy(out_s, out_hbm)

    return body(blob)


def kernel(inputs, table):
    B = inputs.size
    V, D = table.shape
    blob = jnp.concatenate(
        [
            inputs.reshape(-1).astype(jnp.int32),
            lax.bitcast_convert_type(table, jnp.int32).reshape(-1),
        ]
    )
    out = _sc_scalar_lookup(blob, B, V, D)
    return lax.bitcast_convert_type(out, jnp.float32).reshape(
        inputs.shape + (D,)
    )


# speedup vs baseline: 1.0001x; 1.0001x over previous
import functools

import jax
import jax.numpy as jnp
from jax import lax
from jax.experimental import pallas as pl
from jax.experimental.pallas import tpu as pltpu
from jax.experimental.pallas import tpu_sc as plsc


def _sc_scalar_lookup(idx_flat, table_i32, B, V, D):
    mesh = plsc.ScalarSubcoreMesh(axis_name="c", num_cores=1)

    @functools.partial(
        pl.kernel,
        out_type=jax.ShapeDtypeStruct((B * D,), jnp.int32),
        mesh=mesh,
        compiler_params=pltpu.CompilerParams(needs_layout_passes=False),
        scratch_types=[
            pltpu.SMEM((B,), jnp.int32),
            pltpu.SMEM((V * D,), jnp.int32),
            pltpu.SMEM((B * D,), jnp.int32),
            pltpu.SemaphoreType.DMA,
        ],
    )
    def body(idx_hbm, tab_hbm, out_hbm, idx_s, tab_s, out_s, sem):
        ins = [
            pltpu.async_copy(idx_hbm, idx_s, sem),
            pltpu.async_copy(tab_hbm, tab_s, sem),
        ]
        for c in ins:
            c.wait()
        for b in range(B):
            row = idx_s[b]
            for d in range(D):
                out_s[b * D + d] = tab_s[row * D + d]
        pltpu.sync_copy(out_s, out_hbm)

    return body(idx_flat, table_i32)


def kernel(inputs, table):
    B = inputs.size
    V, D = table.shape
    out = _sc_scalar_lookup(
        inputs.reshape(-1).astype(jnp.int32),
        lax.bitcast_convert_type(table, jnp.int32).reshape(-1),
        B,
        V,
        D,
    )
    return lax.bitcast_convert_type(out, jnp.float32).reshape(
        inputs.shape + (D,)
    )


# speedup vs baseline: 1.0792x; 1.0791x over previous
import functools

import jax
import jax.numpy as jnp
from jax import lax
from jax.experimental import pallas as pl
from jax.experimental.pallas import tpu as pltpu
from jax.experimental.pallas import tpu_sc as plsc


def _sc_scalar_lookup(idx_flat, table):
    B = idx_flat.shape[0]
    V, D = table.shape
    mesh = plsc.ScalarSubcoreMesh(axis_name="c", num_cores=1)

    @functools.partial(
        pl.kernel,
        out_type=jax.ShapeDtypeStruct((B, D), jnp.float32),
        mesh=mesh,
        compiler_params=pltpu.CompilerParams(needs_layout_passes=False),
        scratch_types=[
            pltpu.SMEM((B,), jnp.int32),
            pltpu.SMEM((V, D), jnp.float32),
            pltpu.SMEM((B, D), jnp.float32),
            pltpu.SemaphoreType.DMA,
        ],
    )
    def body(idx_hbm, tab_hbm, out_hbm, idx_s, tab_s, out_s, sem):
        ins = [
            pltpu.async_copy(idx_hbm, idx_s, sem),
            pltpu.async_copy(tab_hbm, tab_s, sem),
        ]
        for c in ins:
            c.wait()
        for b in range(B):
            row = idx_s[b]
            for d in range(D):
                out_s[b, d] = tab_s[row, d]
        pltpu.sync_copy(out_s, out_hbm)

    return body(idx_flat, table)


def kernel(inputs, table):
    out = _sc_scalar_lookup(inputs.reshape(-1).astype(jnp.int32), table)
    return out.reshape(inputs.shape + (table.shape[1],))
